# initial kernel scaffold (unmeasured)
import jax
import jax.numpy as jnp
from jax import lax
from jax.experimental import pallas as pl
from jax.experimental.pallas import tpu as pltpu

N_DEV = 32


def kernel(x, w_mat, scale_x, scale_w):
    m_per, k = x.shape
    _, n = w_mat.shape
    n_per = n // N_DEV
    m_total = m_per * N_DEV

    def body(x_ref, w_ref, sx_ref, sw_ref, out_ref, acc_ref, send_sems, recv_sems):
        my_pos = lax.axis_index("i")

        acc = lax.dot_general(
            x_ref[:, :], w_ref[:, :],
            dimension_numbers=(((1,), (0,)), ((), ())),
            preferred_element_type=jnp.float32,
        )
        acc_ref[:, :] = acc * (sx_ref[0] * sw_ref[0])

        sends = []
        for p in range(N_DEV):
            peer = lax.rem(my_pos + p, N_DEV)
            rdma = pltpu.make_async_remote_copy(
                src_ref=acc_ref.at[:, pl.ds(peer * n_per, n_per)],
                dst_ref=out_ref.at[pl.ds(my_pos * m_per, m_per), :],
                send_sem=send_sems.at[p],
                recv_sem=recv_sems.at[p],
                device_id=(peer,),
                device_id_type=pl.DeviceIdType.MESH,
            )
            rdma.start()
            sends.append(rdma)

        for p in range(N_DEV):
            src_dev = lax.rem(my_pos - p + N_DEV, N_DEV)
            recv = pltpu.make_async_remote_copy(
                src_ref=acc_ref.at[:, pl.ds(0, n_per)],
                dst_ref=out_ref.at[pl.ds(src_dev * m_per, m_per), :],
                send_sem=send_sems.at[p],
                recv_sem=recv_sems.at[p],
                device_id=(src_dev,),
                device_id_type=pl.DeviceIdType.MESH,
            )
            recv.wait_recv()

        for rdma in sends:
            rdma.wait_send()

    out_shape = jax.ShapeDtypeStruct((m_total, n_per), jnp.float32)
    return pl.pallas_call(
        body,
        out_shape=out_shape,
        in_specs=[
            pl.BlockSpec(memory_space=pltpu.VMEM),
            pl.BlockSpec(memory_space=pltpu.VMEM),
            pl.BlockSpec(memory_space=pltpu.SMEM),
            pl.BlockSpec(memory_space=pltpu.SMEM),
        ],
        out_specs=pl.BlockSpec(memory_space=pltpu.VMEM),
        scratch_shapes=[
            pltpu.VMEM((m_per, n), jnp.float32),
            pltpu.SemaphoreType.DMA((N_DEV,)),
            pltpu.SemaphoreType.DMA((N_DEV,)),
        ],
        compiler_params=pltpu.CompilerParams(collective_id=0),
    )(x, w_mat, scale_x, scale_w)


# baseline (device time: 44083 ns/iter reference)
import jax
import jax.numpy as jnp
from jax import lax
from jax.experimental import pallas as pl
from jax.experimental.pallas import tpu as pltpu

N_DEV = 32


def kernel(x, w_mat, scale_x, scale_w):
    m_per, k = x.shape
    _, n = w_mat.shape
    n_per = n // N_DEV
    m_total = m_per * N_DEV

    def body(x_ref, w_ref, sx_ref, sw_ref, out_ref,
             acc_t_ref, out_t_ref, send_sems, recv_sems):
        my_pos = lax.axis_index("i")

        acc = lax.dot_general(
            x_ref[:, :], w_ref[:, :],
            dimension_numbers=(((1,), (0,)), ((), ())),
            preferred_element_type=jnp.float32,
        )
        acc_t_ref[:, :] = jnp.transpose(acc * (sx_ref[0] * sw_ref[0]))

        sends = []
        for p in range(N_DEV):
            peer = lax.rem(my_pos + p, N_DEV)
            rdma = pltpu.make_async_remote_copy(
                src_ref=acc_t_ref.at[pl.ds(peer * n_per, n_per), :],
                dst_ref=out_t_ref.at[:, pl.ds(my_pos * m_per, m_per)],
                send_sem=send_sems.at[p],
                recv_sem=recv_sems.at[p],
                device_id=(peer,),
                device_id_type=pl.DeviceIdType.MESH,
            )
            rdma.start()
            sends.append(rdma)

        for p in range(N_DEV):
            src_dev = lax.rem(my_pos - p + N_DEV, N_DEV)
            recv = pltpu.make_async_remote_copy(
                src_ref=acc_t_ref.at[pl.ds(0, n_per), :],
                dst_ref=out_t_ref.at[:, pl.ds(src_dev * m_per, m_per)],
                send_sem=send_sems.at[p],
                recv_sem=recv_sems.at[p],
                device_id=(src_dev,),
                device_id_type=pl.DeviceIdType.MESH,
            )
            recv.wait_recv()

        out_ref[:, :] = jnp.transpose(out_t_ref[:, :])

        for rdma in sends:
            rdma.wait_send()

    out_shape = jax.ShapeDtypeStruct((m_total, n_per), jnp.float32)
    return pl.pallas_call(
        body,
        out_shape=out_shape,
        in_specs=[
            pl.BlockSpec(memory_space=pltpu.VMEM),
            pl.BlockSpec(memory_space=pltpu.VMEM),
            pl.BlockSpec(memory_space=pltpu.SMEM),
            pl.BlockSpec(memory_space=pltpu.SMEM),
        ],
        out_specs=pl.BlockSpec(memory_space=pltpu.VMEM),
        scratch_shapes=[
            pltpu.VMEM((n, m_per), jnp.float32),
            pltpu.VMEM((n_per, m_total), jnp.float32),
            pltpu.SemaphoreType.DMA((N_DEV,)),
            pltpu.SemaphoreType.DMA((N_DEV,)),
        ],
        compiler_params=pltpu.CompilerParams(
            vmem_limit_bytes=56 * 1024 * 1024,
        ),
    )(x, w_mat, scale_x, scale_w)


# device time: 27056 ns/iter; 1.6293x vs baseline; 1.6293x over previous
import os

import jax
import jax.numpy as jnp
from jax import lax
from jax.experimental import pallas as pl
from jax.experimental.pallas import tpu as pltpu

N_DEV = 32
N_CHUNK = 4
_BENCH = os.environ.get("BENCH_MODE", "full")


def kernel(x, w_mat, scale_x, scale_w):
    m_per, k = x.shape
    _, n = w_mat.shape
    n_per = n // N_DEV
    m_total = m_per * N_DEV
    n_chunk = n // N_CHUNK
    peers_per_chunk = N_DEV // N_CHUNK

    def body(x_hbm_ref, w_hbm_ref, sx_ref, sw_ref, out_ref,
             x_ref, w_ref, acc_t_ref, out_t_ref,
             x_sem, w_sems, send_sems, recv_sems):
        my_pos = lax.axis_index("i")
        base = lax.rem(my_pos, N_CHUNK)

        x_dma = pltpu.make_async_copy(x_hbm_ref, x_ref, x_sem)
        x_dma.start()
        w_dmas = []
        for j in range(N_CHUNK):
            c = lax.rem(base + j, N_CHUNK)
            d = pltpu.make_async_copy(
                w_hbm_ref.at[:, pl.ds(c * n_chunk, n_chunk)],
                w_ref.at[:, pl.ds(c * n_chunk, n_chunk)],
                w_sems.at[j],
            )
            d.start()
            w_dmas.append(d)

        barrier_sem = pltpu.get_barrier_semaphore()
        for q in range(1, N_DEV):
            pl.semaphore_signal(
                barrier_sem, inc=1,
                device_id=(lax.rem(my_pos + q, N_DEV),),
                device_id_type=pl.DeviceIdType.MESH,
            )
        pl.semaphore_wait(barrier_sem, N_DEV - 1)
        x_dma.wait()

        scale = sx_ref[0] * sw_ref[0]
        sends = []
        for j in range(N_CHUNK):
            c = lax.rem(base + j, N_CHUNK)
            w_dmas[j].wait()
            acc = lax.dot_general(
                x_ref[:, :], w_ref[:, pl.ds(c * n_chunk, n_chunk)],
                dimension_numbers=(((1,), (0,)), ((), ())),
                preferred_element_type=jnp.float32,
                precision=lax.Precision.DEFAULT,
            )
            acc_t_ref[pl.ds(c * n_chunk, n_chunk), :] = jnp.transpose(
                acc * scale).astype(jnp.bfloat16)
            if _BENCH == "nosend":
                continue
            for b in range(peers_per_chunk):
                peer = c * peers_per_chunk + b
                slot = lax.rem(peer - my_pos + N_DEV, N_DEV)
                rdma = pltpu.make_async_remote_copy(
                    src_ref=acc_t_ref.at[pl.ds(peer * n_per, n_per), :],
                    dst_ref=out_t_ref.at[:, pl.ds(my_pos * m_per, m_per)],
                    send_sem=send_sems.at[slot],
                    recv_sem=recv_sems.at[slot],
                    device_id=(peer,),
                    device_id_type=pl.DeviceIdType.MESH,
                )
                rdma.start()
                sends.append(rdma)

        if _BENCH == "nosend":
            out_ref[:, :] = jnp.zeros((m_total, n_per), jnp.float32)
            return

        for p in range(N_DEV):
            src_dev = lax.rem(my_pos - p + N_DEV, N_DEV)
            recv = pltpu.make_async_remote_copy(
                src_ref=acc_t_ref.at[pl.ds(0, n_per), :],
                dst_ref=out_t_ref.at[:, pl.ds(src_dev * m_per, m_per)],
                send_sem=send_sems.at[p],
                recv_sem=recv_sems.at[p],
                device_id=(src_dev,),
                device_id_type=pl.DeviceIdType.MESH,
            )
            recv.wait_recv()

        out_ref[:, :] = jnp.transpose(out_t_ref[:, :].astype(jnp.float32))

        for rdma in sends:
            rdma.wait_send()

    out_shape = jax.ShapeDtypeStruct((m_total, n_per), jnp.float32)
    return pl.pallas_call(
        body,
        out_shape=out_shape,
        in_specs=[
            pl.BlockSpec(memory_space=pl.ANY),
            pl.BlockSpec(memory_space=pl.ANY),
            pl.BlockSpec(memory_space=pltpu.SMEM),
            pl.BlockSpec(memory_space=pltpu.SMEM),
        ],
        out_specs=pl.BlockSpec(memory_space=pltpu.VMEM),
        scratch_shapes=[
            pltpu.VMEM((m_per, k), jnp.float32),
            pltpu.VMEM((k, n), jnp.float32),
            pltpu.VMEM((n, m_per), jnp.bfloat16),
            pltpu.VMEM((n_per, m_total), jnp.bfloat16),
            pltpu.SemaphoreType.DMA,
            pltpu.SemaphoreType.DMA((N_CHUNK,)),
            pltpu.SemaphoreType.DMA((N_DEV,)),
            pltpu.SemaphoreType.DMA((N_DEV,)),
        ],
        compiler_params=pltpu.CompilerParams(
            collective_id=0,
            vmem_limit_bytes=56 * 1024 * 1024,
        ),
    )(x, w_mat, scale_x, scale_w)
